# Initial kernel scaffold; baseline (speedup 1.0000x reference)
import jax
import jax.numpy as jnp
from jax.experimental import pallas as pl


def kernel(x, W_self1, W_neigh1, b1, W_self2, W_neigh2, b2, edge_index):
    raise NotImplementedError("write your pallas kernel here")


# speedup vs baseline: 3.4954x; 3.4954x over previous
import functools

import jax
import jax.numpy as jnp
from jax import lax
from jax.experimental import pallas as pl
from jax.experimental.pallas import tpu as pltpu
from jax.experimental.pallas import tpu_sc as plsc

N = 10000
E = 160000
D = 256
DC = 64
NCH = 2
NQ = 4
NC = 2
NS = 16
EPT = E // NS
B = 80
G = EPT // B
DEGW = 16
NP = 10240
RPT = NP // NS
ZR = 128


@functools.cache
def _mesh():
    return plsc.VectorSubcoreMesh(core_axis_name="c", subcore_axis_name="s",
                                  num_cores=NC, num_subcores=NS)


def _sc_agg_body(with_deg, xs_hbm, src_hbm, dst_hbm, za_hbm, zd_hbm, on_hbm,
                 agg_hbm, deg_hbm, src_v, dst_v, rows_v, ones_v, agg_s,
                 deg_s, sem):
    c = lax.axis_index("c")
    s = lax.axis_index("s")

    pltpu.sync_copy(dst_hbm.at[s], dst_v)
    if with_deg:
        @pl.when(c == 0)
        def _():
            pltpu.sync_copy(zd_hbm, deg_s.at[pl.ds(s * RPT, RPT)])
            pltpu.sync_copy(on_hbm, ones_v)

    for ch in range(NCH):
        q = NCH * c + ch
        pltpu.sync_copy(src_hbm.at[q * NS + s], src_v)
        for j in range(RPT // ZR):
            pltpu.sync_copy(za_hbm, agg_s.at[pl.ds(s * RPT + j * ZR, ZR)])
        plsc.subcore_barrier()

        deg_pass = with_deg and ch == 0

        def step(g, _, deg_pass=deg_pass):
            src_i = src_v.at[g]
            dst_i = dst_v.at[g]
            pltpu.async_copy(xs_hbm.at[src_i], rows_v, sem).wait()
            pltpu.sync_copy(rows_v, agg_s.at[dst_i], add=True)
            if deg_pass:
                @pl.when(c == 0)
                def _():
                    pltpu.sync_copy(ones_v, deg_s.at[dst_i], add=True)
            return _

        lax.fori_loop(0, G, step, None)
        plsc.subcore_barrier()

        pltpu.sync_copy(agg_s.at[pl.ds(s * RPT, RPT)],
                        agg_hbm.at[pl.ds(q * NP + s * RPT, RPT)])
        if ch + 1 < NCH:
            plsc.subcore_barrier()

    if with_deg:
        @pl.when(jnp.logical_and(c == 0, s == 0))
        def _():
            pltpu.sync_copy(deg_s.at[pl.ds(0, N)], deg_hbm)


@functools.cache
def _make_sc_agg(with_deg):
    out_type = [jax.ShapeDtypeStruct((NQ * NP, DC), jnp.float32)]
    if with_deg:
        out_type.append(jax.ShapeDtypeStruct((N, DEGW), jnp.float32))
    scratch = [
        pltpu.VMEM((G, B), jnp.int32),
        pltpu.VMEM((G, B), jnp.int32),
        pltpu.VMEM((B, DC), jnp.float32),
        pltpu.VMEM((B, DEGW), jnp.float32),
        pltpu.VMEM_SHARED((NP, DC), jnp.float32),
        pltpu.VMEM_SHARED((NP, DEGW), jnp.float32),
        pltpu.SemaphoreType.DMA,
    ]

    def body(*refs):
        if with_deg:
            (xs, src, dst, za, zd, on, agg, deg,
             src_v, dst_v, rows_v, ones_v, agg_s, deg_s, sem) = refs
        else:
            (xs, src, dst, za, zd, on, agg,
             src_v, dst_v, rows_v, ones_v, agg_s, deg_s, sem) = refs
            deg = None
        _sc_agg_body(with_deg, xs, src, dst, za, zd, on, agg, deg,
                     src_v, dst_v, rows_v, ones_v, agg_s, deg_s, sem)

    return pl.kernel(body, out_type=out_type, mesh=_mesh(),
                     scratch_types=scratch,
                     compiler_params=pltpu.CompilerParams(
                         use_tc_tiling_on_sc=False),
                     name="sc_agg_deg" if with_deg else "sc_agg")


RB = 1000


def _neigh_term(agg_ref, wn_ref):
    r = jnp.dot(agg_ref[0], wn_ref[0 * DC:1 * DC, :],
                preferred_element_type=jnp.float32)
    for q in range(1, NQ):
        r += jnp.dot(agg_ref[q], wn_ref[q * DC:(q + 1) * DC, :],
                     preferred_element_type=jnp.float32)
    return r


def _tc_layer1_body(x_ref, agg_ref, deg_ref, ws_ref, wn_ref, b_ref,
                    h_ref, hs_ref):
    r = _neigh_term(agg_ref, wn_ref)
    deg = jnp.maximum(deg_ref[:, 0:1], 1.0)
    h = jnp.dot(x_ref[...], ws_ref[...], preferred_element_type=jnp.float32)
    h = h + r / deg + b_ref[...][None, :]
    h = jnp.maximum(h, 0.0)
    h_ref[...] = h
    for q in range(NQ):
        hs_ref[q] = h[:, q * DC:(q + 1) * DC]


def _tc_layer2_body(h_ref, agg_ref, deg_ref, ws_ref, wn_ref, b_ref, out_ref):
    r = _neigh_term(agg_ref, wn_ref)
    deg = jnp.maximum(deg_ref[:, 0:1], 1.0)
    z = jnp.dot(h_ref[...], ws_ref[...], preferred_element_type=jnp.float32)
    z = z + r / deg + b_ref[...][None, :]
    m = jnp.max(z, axis=1, keepdims=True)
    zm = z - m
    out_ref[...] = zm - jnp.log(jnp.sum(jnp.exp(zm), axis=1, keepdims=True))


_common_in_specs = [
    pl.BlockSpec((RB, D), lambda i: (i, 0)),
    pl.BlockSpec((NQ, RB, DC), lambda i: (0, i, 0)),
    pl.BlockSpec((RB, DEGW), lambda i: (i, 0)),
    pl.BlockSpec((D, D), lambda i: (0, 0)),
    pl.BlockSpec((D, D), lambda i: (0, 0)),
    pl.BlockSpec((D,), lambda i: (0,)),
]


def _tc_layer1(x, agg4, deg16, Ws, Wn, b):
    return pl.pallas_call(
        _tc_layer1_body,
        grid=(N // RB,),
        in_specs=_common_in_specs,
        out_specs=[
            pl.BlockSpec((RB, D), lambda i: (i, 0)),
            pl.BlockSpec((NQ, RB, DC), lambda i: (0, i, 0)),
        ],
        out_shape=[
            jax.ShapeDtypeStruct((N, D), jnp.float32),
            jax.ShapeDtypeStruct((NQ, N, DC), jnp.float32),
        ],
        name="tc_layer1",
    )(x, agg4, deg16, Ws, Wn, b)


def _tc_layer2(h, agg4, deg16, Ws, Wn, b):
    return pl.pallas_call(
        _tc_layer2_body,
        grid=(N // RB,),
        in_specs=_common_in_specs,
        out_specs=pl.BlockSpec((RB, D), lambda i: (i, 0)),
        out_shape=jax.ShapeDtypeStruct((N, D), jnp.float32),
        name="tc_layer2",
    )(h, agg4, deg16, Ws, Wn, b)


def kernel(x, W_self1, W_neigh1, b1, W_self2, W_neigh2, b2, edge_index):
    src = edge_index[0].astype(jnp.int32)
    dst = edge_index[1].astype(jnp.int32)
    srcr = src.reshape(NS, G, B)
    src4 = jnp.concatenate([srcr + q * N for q in range(NQ)], axis=0)
    dstr = dst.reshape(NS, G, B)

    xsplit = x.reshape(N, NQ, DC).transpose(1, 0, 2).reshape(NQ * N, DC)

    zeros_a = jnp.zeros((ZR, DC), jnp.float32)
    zeros_d = jnp.zeros((RPT, DEGW), jnp.float32)
    ones_d = jnp.ones((B, DEGW), jnp.float32)

    agg1, deg16 = _make_sc_agg(True)(xsplit, src4, dstr, zeros_a, zeros_d,
                                     ones_d)
    h, hsplit = _tc_layer1(x, agg1.reshape(NQ, NP, DC), deg16,
                           W_self1, W_neigh1, b1)
    (agg2,) = _make_sc_agg(False)(hsplit.reshape(NQ * N, DC), src4, dstr,
                                  zeros_a, zeros_d, ones_d)
    return _tc_layer2(h, agg2.reshape(NQ, NP, DC), deg16,
                      W_self2, W_neigh2, b2)


# speedup vs baseline: 5.3017x; 1.5168x over previous
import functools

import jax
import jax.numpy as jnp
from jax import lax
from jax.experimental import pallas as pl
from jax.experimental.pallas import tpu as pltpu
from jax.experimental.pallas import tpu_sc as plsc

N = 10000
E = 160000
D = 256
DC = 64
NCH = 2
NQ = 4
NC = 2
NS = 16
EPT = E // NS
B = 125
G = EPT // B
DEGW = 16
NP = 10240
RPT = NP // NS
ZR = 128


@functools.cache
def _mesh():
    return plsc.VectorSubcoreMesh(core_axis_name="c", subcore_axis_name="s",
                                  num_cores=NC, num_subcores=NS)


def _sc_agg_body(with_deg, xs_hbm, src_hbm, dst_hbm, za_hbm, zd_hbm, on_hbm,
                 agg_hbm, deg_hbm, src_v, dst_v, rows0, rows1, ones_v, agg_s,
                 deg_s, sg0, sg1, ss0, ss1, sd):
    c = lax.axis_index("c")
    s = lax.axis_index("s")

    pltpu.sync_copy(dst_hbm.at[s], dst_v)
    if with_deg:
        @pl.when(c == 0)
        def _():
            pltpu.sync_copy(zd_hbm, deg_s.at[pl.ds(s * RPT, RPT)])
            pltpu.sync_copy(on_hbm, ones_v)

    def gather(g, rows, sem):
        return pltpu.make_async_copy(xs_hbm.at[src_v.at[g]], rows, sem)

    def scatter(g, rows, sem):
        return pltpu.make_async_copy(rows, agg_s.at[dst_v.at[g]], sem)

    for ch in range(NCH):
        q = NCH * c + ch
        pltpu.sync_copy(src_hbm.at[q * NS + s], src_v)
        for j in range(RPT // ZR):
            pltpu.sync_copy(za_hbm, agg_s.at[pl.ds(s * RPT + j * ZR, ZR)])
        plsc.subcore_barrier()

        deg_pass = with_deg and ch == 0

        gather(0, rows0, sg0).start()
        gather(1, rows1, sg1).start()

        def pair(i, _, deg_pass=deg_pass):
            g0 = 2 * i
            g1 = g0 + 1
            gather(g0, rows0, sg0).wait()
            scatter(g0, rows0, ss0).start(add=True)
            if deg_pass:
                @pl.when(c == 0)
                def _():
                    @pl.when(i > 0)
                    def _():
                        pltpu.make_async_copy(
                            ones_v, deg_s.at[dst_v.at[g0]], sd).wait()
                        pltpu.make_async_copy(
                            ones_v, deg_s.at[dst_v.at[g0]], sd).wait()
                    pltpu.make_async_copy(
                        ones_v, deg_s.at[dst_v.at[g0]], sd).start(add=True)
                    pltpu.make_async_copy(
                        ones_v, deg_s.at[dst_v.at[g1]], sd).start(add=True)
            gather(g1, rows1, sg1).wait()
            scatter(g1, rows1, ss1).start(add=True)

            @pl.when(i + 1 < G // 2)
            def _():
                scatter(g0, rows0, ss0).wait()
                gather(g0 + 2, rows0, sg0).start()
                scatter(g1, rows1, ss1).wait()
                gather(g1 + 2, rows1, sg1).start()
            return _

        lax.fori_loop(0, G // 2, pair, None)
        scatter(G - 2, rows0, ss0).wait()
        scatter(G - 1, rows1, ss1).wait()
        if deg_pass:
            @pl.when(c == 0)
            def _():
                pltpu.make_async_copy(ones_v, deg_s.at[dst_v.at[0]],
                                      sd).wait()
                pltpu.make_async_copy(ones_v, deg_s.at[dst_v.at[0]],
                                      sd).wait()
        plsc.subcore_barrier()

        pltpu.sync_copy(agg_s.at[pl.ds(s * RPT, RPT)],
                        agg_hbm.at[pl.ds(q * NP + s * RPT, RPT)])
        if ch + 1 < NCH:
            plsc.subcore_barrier()

    if with_deg:
        @pl.when(jnp.logical_and(c == 0, s == 0))
        def _():
            pltpu.sync_copy(deg_s.at[pl.ds(0, N)], deg_hbm)


@functools.cache
def _make_sc_agg(with_deg):
    out_type = [jax.ShapeDtypeStruct((NQ * NP, DC), jnp.float32)]
    if with_deg:
        out_type.append(jax.ShapeDtypeStruct((N, DEGW), jnp.float32))
    scratch = [
        pltpu.VMEM((G, B), jnp.int32),
        pltpu.VMEM((G, B), jnp.int32),
        pltpu.VMEM((B, DC), jnp.float32),
        pltpu.VMEM((B, DC), jnp.float32),
        pltpu.VMEM((B, DEGW), jnp.float32),
        pltpu.VMEM_SHARED((NP, DC), jnp.float32),
        pltpu.VMEM_SHARED((NP, DEGW), jnp.float32),
        pltpu.SemaphoreType.DMA,
        pltpu.SemaphoreType.DMA,
        pltpu.SemaphoreType.DMA,
        pltpu.SemaphoreType.DMA,
        pltpu.SemaphoreType.DMA,
    ]

    def body(*refs):
        if with_deg:
            (xs, src, dst, za, zd, on, agg, deg, *rest) = refs
        else:
            (xs, src, dst, za, zd, on, agg, *rest) = refs
            deg = None
        _sc_agg_body(with_deg, xs, src, dst, za, zd, on, agg, deg, *rest)

    return pl.kernel(body, out_type=out_type, mesh=_mesh(),
                     scratch_types=scratch,
                     compiler_params=pltpu.CompilerParams(
                         use_tc_tiling_on_sc=False),
                     name="sc_agg_deg" if with_deg else "sc_agg")


RB = 1000


def _neigh_term(agg_ref, wn_ref):
    r = jnp.dot(agg_ref[0], wn_ref[0 * DC:1 * DC, :],
                preferred_element_type=jnp.float32)
    for q in range(1, NQ):
        r += jnp.dot(agg_ref[q], wn_ref[q * DC:(q + 1) * DC, :],
                     preferred_element_type=jnp.float32)
    return r


def _tc_layer1_body(x_ref, agg_ref, deg_ref, ws_ref, wn_ref, b_ref,
                    h_ref, hs_ref):
    r = _neigh_term(agg_ref, wn_ref)
    deg = jnp.maximum(deg_ref[:, 0:1], 1.0)
    h = jnp.dot(x_ref[...], ws_ref[...], preferred_element_type=jnp.float32)
    h = h + r / deg + b_ref[...][None, :]
    h = jnp.maximum(h, 0.0)
    h_ref[...] = h
    for q in range(NQ):
        hs_ref[q] = h[:, q * DC:(q + 1) * DC]


def _tc_layer2_body(h_ref, agg_ref, deg_ref, ws_ref, wn_ref, b_ref, out_ref):
    r = _neigh_term(agg_ref, wn_ref)
    deg = jnp.maximum(deg_ref[:, 0:1], 1.0)
    z = jnp.dot(h_ref[...], ws_ref[...], preferred_element_type=jnp.float32)
    z = z + r / deg + b_ref[...][None, :]
    m = jnp.max(z, axis=1, keepdims=True)
    zm = z - m
    out_ref[...] = zm - jnp.log(jnp.sum(jnp.exp(zm), axis=1, keepdims=True))


_common_in_specs = [
    pl.BlockSpec((RB, D), lambda i: (i, 0)),
    pl.BlockSpec((NQ, RB, DC), lambda i: (0, i, 0)),
    pl.BlockSpec((RB, DEGW), lambda i: (i, 0)),
    pl.BlockSpec((D, D), lambda i: (0, 0)),
    pl.BlockSpec((D, D), lambda i: (0, 0)),
    pl.BlockSpec((D,), lambda i: (0,)),
]


def _tc_layer1(x, agg4, deg16, Ws, Wn, b):
    return pl.pallas_call(
        _tc_layer1_body,
        grid=(N // RB,),
        in_specs=_common_in_specs,
        out_specs=[
            pl.BlockSpec((RB, D), lambda i: (i, 0)),
            pl.BlockSpec((NQ, RB, DC), lambda i: (0, i, 0)),
        ],
        out_shape=[
            jax.ShapeDtypeStruct((N, D), jnp.float32),
            jax.ShapeDtypeStruct((NQ, N, DC), jnp.float32),
        ],
        name="tc_layer1",
    )(x, agg4, deg16, Ws, Wn, b)


def _tc_layer2(h, agg4, deg16, Ws, Wn, b):
    return pl.pallas_call(
        _tc_layer2_body,
        grid=(N // RB,),
        in_specs=_common_in_specs,
        out_specs=pl.BlockSpec((RB, D), lambda i: (i, 0)),
        out_shape=jax.ShapeDtypeStruct((N, D), jnp.float32),
        name="tc_layer2",
    )(h, agg4, deg16, Ws, Wn, b)


def kernel(x, W_self1, W_neigh1, b1, W_self2, W_neigh2, b2, edge_index):
    src = edge_index[0].astype(jnp.int32)
    dst = edge_index[1].astype(jnp.int32)
    srcr = src.reshape(NS, G, B)
    src4 = jnp.concatenate([srcr + q * N for q in range(NQ)], axis=0)
    dstr = dst.reshape(NS, G, B)

    xsplit = x.reshape(N, NQ, DC).transpose(1, 0, 2).reshape(NQ * N, DC)

    zeros_a = jnp.zeros((ZR, DC), jnp.float32)
    zeros_d = jnp.zeros((RPT, DEGW), jnp.float32)
    ones_d = jnp.ones((B, DEGW), jnp.float32)

    agg1, deg16 = _make_sc_agg(True)(xsplit, src4, dstr, zeros_a, zeros_d,
                                     ones_d)
    h, hsplit = _tc_layer1(x, agg1.reshape(NQ, NP, DC), deg16,
                           W_self1, W_neigh1, b1)
    (agg2,) = _make_sc_agg(False)(hsplit.reshape(NQ * N, DC), src4, dstr,
                                  zeros_a, zeros_d, ones_d)
    return _tc_layer2(h, agg2.reshape(NQ, NP, DC), deg16,
                      W_self2, W_neigh2, b2)


# speedup vs baseline: 6.4281x; 1.2125x over previous
import functools

import jax
import jax.numpy as jnp
from jax import lax
from jax.experimental import pallas as pl
from jax.experimental.pallas import tpu as pltpu
from jax.experimental.pallas import tpu_sc as plsc

N = 10000
E = 160000
D = 256
DC = 64
NCH = 2
NQ = 4
NC = 2
NS = 16
EPT = E // NS
B = 125
G = EPT // B
DEGW = 16
NP = 10240
RPT = NP // NS
ZR = 128


@functools.cache
def _mesh():
    return plsc.VectorSubcoreMesh(core_axis_name="c", subcore_axis_name="s",
                                  num_cores=NC, num_subcores=NS)


NBUF = 4


def _sc_agg_body(with_deg, xs_hbm, src_hbm, dst_hbm, za_hbm, zd_hbm, on_hbm,
                 agg_hbm, deg_hbm, src_v, dst_v, *rest):
    rows = rest[:NBUF]
    ones_v, agg_s, deg_s = rest[NBUF:NBUF + 3]
    sg = rest[NBUF + 3:2 * NBUF + 3]
    ss = rest[2 * NBUF + 3:3 * NBUF + 3]
    sd = rest[3 * NBUF + 3]
    c = lax.axis_index("c")
    s = lax.axis_index("s")

    pltpu.sync_copy(dst_hbm.at[s], dst_v)
    if with_deg:
        @pl.when(c == 0)
        def _():
            pltpu.sync_copy(zd_hbm, deg_s.at[pl.ds(s * RPT, RPT)])
            pltpu.sync_copy(on_hbm, ones_v)

    def gather(g, rows, sem):
        return pltpu.make_async_copy(xs_hbm.at[src_v.at[g]], rows, sem)

    def scatter(g, rows, sem):
        return pltpu.make_async_copy(rows, agg_s.at[dst_v.at[g]], sem)

    for ch in range(NCH):
        q = NCH * c + ch
        pltpu.sync_copy(src_hbm.at[q * NS + s], src_v)
        for j in range(RPT // ZR):
            pltpu.sync_copy(za_hbm, agg_s.at[pl.ds(s * RPT + j * ZR, ZR)])
        plsc.subcore_barrier()

        deg_pass = with_deg and ch == 0

        def deg_add(g):
            return pltpu.make_async_copy(ones_v, deg_s.at[dst_v.at[g]], sd)

        for k in range(NBUF):
            gather(k, rows[k], sg[k]).start()

        def stage(i, _, deg_pass=deg_pass):
            base = NBUF * i
            for k in range(NBUF):
                g = base + k
                gather(g, rows[k], sg[k]).wait()
                scatter(g, rows[k], ss[k]).start(add=True)
            if deg_pass:
                @pl.when(c == 0)
                def _():
                    @pl.when(i > 0)
                    def _():
                        for k in range(NBUF):
                            deg_add(base + k).wait()
                    for k in range(NBUF):
                        deg_add(base + k).start(add=True)

            @pl.when(i + 1 < G // NBUF)
            def _():
                for k in range(NBUF):
                    scatter(base + k, rows[k], ss[k]).wait()
                    gather(base + NBUF + k, rows[k], sg[k]).start()
            return _

        lax.fori_loop(0, G // NBUF, stage, None)
        for k in range(NBUF):
            scatter(G - NBUF + k, rows[k], ss[k]).wait()
        if deg_pass:
            @pl.when(c == 0)
            def _():
                for k in range(NBUF):
                    deg_add(0).wait()
        plsc.subcore_barrier()

        pltpu.sync_copy(agg_s.at[pl.ds(s * RPT, RPT)],
                        agg_hbm.at[pl.ds(q * NP + s * RPT, RPT)])
        if ch + 1 < NCH:
            plsc.subcore_barrier()

    if with_deg:
        @pl.when(jnp.logical_and(c == 0, s == 0))
        def _():
            pltpu.sync_copy(deg_s.at[pl.ds(0, N)], deg_hbm)


@functools.cache
def _make_sc_agg(with_deg):
    out_type = [jax.ShapeDtypeStruct((NQ * NP, DC), jnp.float32)]
    if with_deg:
        out_type.append(jax.ShapeDtypeStruct((N, DEGW), jnp.float32))
    scratch = (
        [pltpu.VMEM((G, B), jnp.int32)] * 2
        + [pltpu.VMEM((B, DC), jnp.float32)] * NBUF
        + [pltpu.VMEM((B, DEGW), jnp.float32)]
        + [pltpu.VMEM_SHARED((NP, DC), jnp.float32)]
        + [pltpu.VMEM_SHARED((NP, DEGW), jnp.float32)]
        + [pltpu.SemaphoreType.DMA] * (2 * NBUF + 1)
    )

    def body(*refs):
        if with_deg:
            (xs, src, dst, za, zd, on, agg, deg, *rest) = refs
        else:
            (xs, src, dst, za, zd, on, agg, *rest) = refs
            deg = None
        _sc_agg_body(with_deg, xs, src, dst, za, zd, on, agg, deg, *rest)

    return pl.kernel(body, out_type=out_type, mesh=_mesh(),
                     scratch_types=scratch,
                     compiler_params=pltpu.CompilerParams(
                         use_tc_tiling_on_sc=False),
                     name="sc_agg_deg" if with_deg else "sc_agg")


RB = 1000


def _neigh_term(agg_ref, wn_ref):
    r = jnp.dot(agg_ref[0], wn_ref[0 * DC:1 * DC, :],
                preferred_element_type=jnp.float32)
    for q in range(1, NQ):
        r += jnp.dot(agg_ref[q], wn_ref[q * DC:(q + 1) * DC, :],
                     preferred_element_type=jnp.float32)
    return r


def _tc_layer1_body(x_ref, agg_ref, deg_ref, ws_ref, wn_ref, b_ref,
                    h_ref, hs_ref):
    r = _neigh_term(agg_ref, wn_ref)
    deg = jnp.maximum(deg_ref[:, 0:1], 1.0)
    h = jnp.dot(x_ref[...], ws_ref[...], preferred_element_type=jnp.float32)
    h = h + r / deg + b_ref[...][None, :]
    h = jnp.maximum(h, 0.0)
    h_ref[...] = h
    for q in range(NQ):
        hs_ref[q] = h[:, q * DC:(q + 1) * DC]


def _tc_layer2_body(h_ref, agg_ref, deg_ref, ws_ref, wn_ref, b_ref, out_ref):
    r = _neigh_term(agg_ref, wn_ref)
    deg = jnp.maximum(deg_ref[:, 0:1], 1.0)
    z = jnp.dot(h_ref[...], ws_ref[...], preferred_element_type=jnp.float32)
    z = z + r / deg + b_ref[...][None, :]
    m = jnp.max(z, axis=1, keepdims=True)
    zm = z - m
    out_ref[...] = zm - jnp.log(jnp.sum(jnp.exp(zm), axis=1, keepdims=True))


_common_in_specs = [
    pl.BlockSpec((RB, D), lambda i: (i, 0)),
    pl.BlockSpec((NQ, RB, DC), lambda i: (0, i, 0)),
    pl.BlockSpec((RB, DEGW), lambda i: (i, 0)),
    pl.BlockSpec((D, D), lambda i: (0, 0)),
    pl.BlockSpec((D, D), lambda i: (0, 0)),
    pl.BlockSpec((D,), lambda i: (0,)),
]


def _tc_layer1(x, agg4, deg16, Ws, Wn, b):
    return pl.pallas_call(
        _tc_layer1_body,
        grid=(N // RB,),
        in_specs=_common_in_specs,
        out_specs=[
            pl.BlockSpec((RB, D), lambda i: (i, 0)),
            pl.BlockSpec((NQ, RB, DC), lambda i: (0, i, 0)),
        ],
        out_shape=[
            jax.ShapeDtypeStruct((N, D), jnp.float32),
            jax.ShapeDtypeStruct((NQ, N, DC), jnp.float32),
        ],
        name="tc_layer1",
    )(x, agg4, deg16, Ws, Wn, b)


def _tc_layer2(h, agg4, deg16, Ws, Wn, b):
    return pl.pallas_call(
        _tc_layer2_body,
        grid=(N // RB,),
        in_specs=_common_in_specs,
        out_specs=pl.BlockSpec((RB, D), lambda i: (i, 0)),
        out_shape=jax.ShapeDtypeStruct((N, D), jnp.float32),
        name="tc_layer2",
    )(h, agg4, deg16, Ws, Wn, b)


def kernel(x, W_self1, W_neigh1, b1, W_self2, W_neigh2, b2, edge_index):
    src = edge_index[0].astype(jnp.int32)
    dst = edge_index[1].astype(jnp.int32)
    srcr = src.reshape(NS, G, B)
    src4 = jnp.concatenate([srcr + q * N for q in range(NQ)], axis=0)
    dstr = dst.reshape(NS, G, B)

    xsplit = x.reshape(N, NQ, DC).transpose(1, 0, 2).reshape(NQ * N, DC)

    zeros_a = jnp.zeros((ZR, DC), jnp.float32)
    zeros_d = jnp.zeros((RPT, DEGW), jnp.float32)
    ones_d = jnp.ones((B, DEGW), jnp.float32)

    agg1, deg16 = _make_sc_agg(True)(xsplit, src4, dstr, zeros_a, zeros_d,
                                     ones_d)
    h, hsplit = _tc_layer1(x, agg1.reshape(NQ, NP, DC), deg16,
                           W_self1, W_neigh1, b1)
    (agg2,) = _make_sc_agg(False)(hsplit.reshape(NQ * N, DC), src4, dstr,
                                  zeros_a, zeros_d, ones_d)
    return _tc_layer2(h, agg2.reshape(NQ, NP, DC), deg16,
                      W_self2, W_neigh2, b2)


# speedup vs baseline: 6.5072x; 1.0123x over previous
import functools

import jax
import jax.numpy as jnp
from jax import lax
from jax.experimental import pallas as pl
from jax.experimental.pallas import tpu as pltpu
from jax.experimental.pallas import tpu_sc as plsc

N = 10000
E = 160000
D = 256
DC = 64
NCH = 2
NQ = 4
NC = 2
NS = 16
EPT = E // NS
B = 125
G = EPT // B
DEGW = 16
NP = 10240
RPT = NP // NS
ZR = 128


@functools.cache
def _mesh():
    return plsc.VectorSubcoreMesh(core_axis_name="c", subcore_axis_name="s",
                                  num_cores=NC, num_subcores=NS)


NBUF = 5


def _sc_agg_body(with_deg, xs_hbm, src_hbm, dst_hbm, za_hbm, zd_hbm, on_hbm,
                 agg_hbm, deg_hbm, src_v, dst_v, *rest):
    rows = rest[:NBUF]
    ones_v, agg_s, deg_s = rest[NBUF:NBUF + 3]
    sg = rest[NBUF + 3:2 * NBUF + 3]
    ss = rest[2 * NBUF + 3:3 * NBUF + 3]
    sd = rest[3 * NBUF + 3]
    c = lax.axis_index("c")
    s = lax.axis_index("s")

    pltpu.sync_copy(dst_hbm.at[s], dst_v)
    if with_deg:
        @pl.when(c == 0)
        def _():
            pltpu.sync_copy(zd_hbm, deg_s.at[pl.ds(s * RPT, RPT)])
            pltpu.sync_copy(on_hbm, ones_v)

    def gather(g, rows, sem):
        return pltpu.make_async_copy(xs_hbm.at[src_v.at[g]], rows, sem)

    def scatter(g, rows, sem):
        return pltpu.make_async_copy(rows, agg_s.at[dst_v.at[g]], sem)

    for ch in range(NCH):
        q = NCH * c + ch
        pltpu.sync_copy(src_hbm.at[q * NS + s], src_v)
        for j in range(RPT // ZR):
            pltpu.sync_copy(za_hbm, agg_s.at[pl.ds(s * RPT + j * ZR, ZR)])
        plsc.subcore_barrier()

        deg_pass = with_deg and ch == 0

        def deg_add(g):
            return pltpu.make_async_copy(ones_v, deg_s.at[dst_v.at[g]], sd)

        for k in range(NBUF):
            gather(k, rows[k], sg[k]).start()

        def stage(i, _, deg_pass=deg_pass):
            base = NBUF * i
            for k in range(NBUF):
                g = base + k
                gather(g, rows[k], sg[k]).wait()
                scatter(g, rows[k], ss[k]).start(add=True)
            if deg_pass:
                @pl.when(c == 0)
                def _():
                    @pl.when(i > 0)
                    def _():
                        for k in range(NBUF):
                            deg_add(base + k).wait()
                    for k in range(NBUF):
                        deg_add(base + k).start(add=True)

            @pl.when(i + 1 < G // NBUF)
            def _():
                for k in range(NBUF):
                    scatter(base + k, rows[k], ss[k]).wait()
                    gather(base + NBUF + k, rows[k], sg[k]).start()
            return _

        lax.fori_loop(0, G // NBUF, stage, None)
        for k in range(NBUF):
            scatter(G - NBUF + k, rows[k], ss[k]).wait()
        if deg_pass:
            @pl.when(c == 0)
            def _():
                for k in range(NBUF):
                    deg_add(0).wait()
        plsc.subcore_barrier()

        pltpu.sync_copy(agg_s.at[pl.ds(s * RPT, RPT)],
                        agg_hbm.at[pl.ds(q * NP + s * RPT, RPT)])
        if ch + 1 < NCH:
            plsc.subcore_barrier()

    if with_deg:
        @pl.when(jnp.logical_and(c == 0, s == 0))
        def _():
            pltpu.sync_copy(deg_s.at[pl.ds(0, N)], deg_hbm)


@functools.cache
def _make_sc_agg(with_deg):
    out_type = [jax.ShapeDtypeStruct((NQ * NP, DC), jnp.float32)]
    if with_deg:
        out_type.append(jax.ShapeDtypeStruct((N, DEGW), jnp.float32))
    scratch = (
        [pltpu.VMEM((G, B), jnp.int32)] * 2
        + [pltpu.VMEM((B, DC), jnp.float32)] * NBUF
        + [pltpu.VMEM((B, DEGW), jnp.float32)]
        + [pltpu.VMEM_SHARED((NP, DC), jnp.float32)]
        + [pltpu.VMEM_SHARED((NP, DEGW), jnp.float32)]
        + [pltpu.SemaphoreType.DMA] * (2 * NBUF + 1)
    )

    def body(*refs):
        if with_deg:
            (xs, src, dst, za, zd, on, agg, deg, *rest) = refs
        else:
            (xs, src, dst, za, zd, on, agg, *rest) = refs
            deg = None
        _sc_agg_body(with_deg, xs, src, dst, za, zd, on, agg, deg, *rest)

    return pl.kernel(body, out_type=out_type, mesh=_mesh(),
                     scratch_types=scratch,
                     compiler_params=pltpu.CompilerParams(
                         use_tc_tiling_on_sc=False),
                     name="sc_agg_deg" if with_deg else "sc_agg")


RB = 1000


def _neigh_term(agg_ref, wn_ref):
    r = jnp.dot(agg_ref[0], wn_ref[0 * DC:1 * DC, :],
                preferred_element_type=jnp.float32)
    for q in range(1, NQ):
        r += jnp.dot(agg_ref[q], wn_ref[q * DC:(q + 1) * DC, :],
                     preferred_element_type=jnp.float32)
    return r


def _tc_layer1_body(x_ref, agg_ref, deg_ref, ws_ref, wn_ref, b_ref,
                    h_ref, hs_ref):
    r = _neigh_term(agg_ref, wn_ref)
    deg = jnp.maximum(deg_ref[:, 0:1], 1.0)
    h = jnp.dot(x_ref[...], ws_ref[...], preferred_element_type=jnp.float32)
    h = h + r / deg + b_ref[...][None, :]
    h = jnp.maximum(h, 0.0)
    h_ref[...] = h
    for q in range(NQ):
        hs_ref[q] = h[:, q * DC:(q + 1) * DC]


def _tc_layer2_body(h_ref, agg_ref, deg_ref, ws_ref, wn_ref, b_ref, out_ref):
    r = _neigh_term(agg_ref, wn_ref)
    deg = jnp.maximum(deg_ref[:, 0:1], 1.0)
    z = jnp.dot(h_ref[...], ws_ref[...], preferred_element_type=jnp.float32)
    z = z + r / deg + b_ref[...][None, :]
    m = jnp.max(z, axis=1, keepdims=True)
    zm = z - m
    out_ref[...] = zm - jnp.log(jnp.sum(jnp.exp(zm), axis=1, keepdims=True))


_common_in_specs = [
    pl.BlockSpec((RB, D), lambda i: (i, 0)),
    pl.BlockSpec((NQ, RB, DC), lambda i: (0, i, 0)),
    pl.BlockSpec((RB, DEGW), lambda i: (i, 0)),
    pl.BlockSpec((D, D), lambda i: (0, 0)),
    pl.BlockSpec((D, D), lambda i: (0, 0)),
    pl.BlockSpec((D,), lambda i: (0,)),
]


def _tc_layer1(x, agg4, deg16, Ws, Wn, b):
    return pl.pallas_call(
        _tc_layer1_body,
        grid=(N // RB,),
        in_specs=_common_in_specs,
        out_specs=[
            pl.BlockSpec((RB, D), lambda i: (i, 0)),
            pl.BlockSpec((NQ, RB, DC), lambda i: (0, i, 0)),
        ],
        out_shape=[
            jax.ShapeDtypeStruct((N, D), jnp.float32),
            jax.ShapeDtypeStruct((NQ, N, DC), jnp.float32),
        ],
        name="tc_layer1",
    )(x, agg4, deg16, Ws, Wn, b)


def _tc_layer2(h, agg4, deg16, Ws, Wn, b):
    return pl.pallas_call(
        _tc_layer2_body,
        grid=(N // RB,),
        in_specs=_common_in_specs,
        out_specs=pl.BlockSpec((RB, D), lambda i: (i, 0)),
        out_shape=jax.ShapeDtypeStruct((N, D), jnp.float32),
        name="tc_layer2",
    )(h, agg4, deg16, Ws, Wn, b)


def kernel(x, W_self1, W_neigh1, b1, W_self2, W_neigh2, b2, edge_index):
    src = edge_index[0].astype(jnp.int32)
    dst = edge_index[1].astype(jnp.int32)
    srcr = src.reshape(NS, G, B)
    src4 = jnp.concatenate([srcr + q * N for q in range(NQ)], axis=0)
    dstr = dst.reshape(NS, G, B)

    xsplit = x.reshape(N, NQ, DC).transpose(1, 0, 2).reshape(NQ * N, DC)

    zeros_a = jnp.zeros((ZR, DC), jnp.float32)
    zeros_d = jnp.zeros((RPT, DEGW), jnp.float32)
    ones_d = jnp.ones((B, DEGW), jnp.float32)

    agg1, deg16 = _make_sc_agg(True)(xsplit, src4, dstr, zeros_a, zeros_d,
                                     ones_d)
    h, hsplit = _tc_layer1(x, agg1.reshape(NQ, NP, DC), deg16,
                           W_self1, W_neigh1, b1)
    (agg2,) = _make_sc_agg(False)(hsplit.reshape(NQ * N, DC), src4, dstr,
                                  zeros_a, zeros_d, ones_d)
    return _tc_layer2(h, agg2.reshape(NQ, NP, DC), deg16,
                      W_self2, W_neigh2, b2)


# speedup vs baseline: 7.6479x; 1.1753x over previous
import functools

import jax
import jax.numpy as jnp
from jax import lax
from jax.experimental import pallas as pl
from jax.experimental.pallas import tpu as pltpu
from jax.experimental.pallas import tpu_sc as plsc

N = 10000
E = 160000
D = 256
DC = 64
NCH = 2
NQ = 4
NC = 2
NS = 16
EPT = E // NS
B = 125
G = EPT // B
DEGW = 16
NP = 10000
RPT = NP // NS
ZR = 125


@functools.cache
def _mesh():
    return plsc.VectorSubcoreMesh(core_axis_name="c", subcore_axis_name="s",
                                  num_cores=NC, num_subcores=NS)


NBUF = 5


def _sc_agg_body(with_deg, xs_hbm, src_hbm, dst_hbm, za_hbm, zd_hbm, on_hbm,
                 agg_hbm, deg_hbm, src_v, dst_v, *rest):
    rows = rest[:NBUF]
    ones_v, agg_s, deg_s = rest[NBUF:NBUF + 3]
    sg = rest[NBUF + 3:2 * NBUF + 3]
    ss = rest[2 * NBUF + 3:3 * NBUF + 3]
    sd = rest[3 * NBUF + 3]
    c = lax.axis_index("c")
    s = lax.axis_index("s")

    pltpu.sync_copy(dst_hbm.at[s], dst_v)
    if with_deg:
        @pl.when(c == 0)
        def _():
            pltpu.sync_copy(zd_hbm, deg_s.at[pl.ds(s * RPT, RPT)])
            pltpu.sync_copy(on_hbm, ones_v)

    def gather(g, rows, sem):
        return pltpu.make_async_copy(xs_hbm.at[src_v.at[g]], rows, sem)

    def scatter(g, rows, sem):
        return pltpu.make_async_copy(rows, agg_s.at[dst_v.at[g]], sem)

    for ch in range(NCH):
        q = NCH * c + ch
        pltpu.sync_copy(src_hbm.at[q * NS + s], src_v)

        deg_pass = with_deg and ch == 0

        def deg_add(g):
            return pltpu.make_async_copy(ones_v, deg_s.at[dst_v.at[g]], sd)

        for k in range(NBUF):
            gather(k, rows[k], sg[k]).start()

        if ch > 0:
            pltpu.sync_copy(agg_s.at[pl.ds(s * RPT, RPT)],
                            agg_hbm.at[c, pl.ds(s * RPT, RPT),
                                       pl.ds((ch - 1) * DC, DC)])
        for j in range(RPT // ZR):
            pltpu.sync_copy(za_hbm, agg_s.at[pl.ds(s * RPT + j * ZR, ZR)])
        plsc.subcore_barrier()

        def stage(i, _, deg_pass=deg_pass):
            base = NBUF * i
            for k in range(NBUF):
                g = base + k
                gather(g, rows[k], sg[k]).wait()
                scatter(g, rows[k], ss[k]).start(add=True)
            if deg_pass:
                @pl.when(c == 0)
                def _():
                    @pl.when(i > 0)
                    def _():
                        for k in range(NBUF):
                            deg_add(base + k).wait()
                    for k in range(NBUF):
                        deg_add(base + k).start(add=True)

            @pl.when(i + 1 < G // NBUF)
            def _():
                for k in range(NBUF):
                    scatter(base + k, rows[k], ss[k]).wait()
                    gather(base + NBUF + k, rows[k], sg[k]).start()
            return _

        lax.fori_loop(0, G // NBUF, stage, None)
        for k in range(NBUF):
            scatter(G - NBUF + k, rows[k], ss[k]).wait()
        if deg_pass:
            @pl.when(c == 0)
            def _():
                for k in range(NBUF):
                    deg_add(0).wait()
        plsc.subcore_barrier()

    pltpu.sync_copy(agg_s.at[pl.ds(s * RPT, RPT)],
                    agg_hbm.at[c, pl.ds(s * RPT, RPT),
                               pl.ds((NCH - 1) * DC, DC)])
    if with_deg:
        @pl.when(jnp.logical_and(c == 0, s == 0))
        def _():
            pltpu.sync_copy(deg_s.at[pl.ds(0, N)], deg_hbm)


@functools.cache
def _make_sc_agg(with_deg):
    out_type = [jax.ShapeDtypeStruct((NC, NP, NCH * DC), jnp.float32)]
    if with_deg:
        out_type.append(jax.ShapeDtypeStruct((N, DEGW), jnp.float32))
    scratch = (
        [pltpu.VMEM((G, B), jnp.int32)] * 2
        + [pltpu.VMEM((B, DC), jnp.float32)] * NBUF
        + [pltpu.VMEM((B, DEGW), jnp.float32)]
        + [pltpu.VMEM_SHARED((NP, DC), jnp.float32)]
        + [pltpu.VMEM_SHARED((NP, DEGW), jnp.float32)]
        + [pltpu.SemaphoreType.DMA] * (2 * NBUF + 1)
    )

    def body(*refs):
        if with_deg:
            (xs, src, dst, za, zd, on, agg, deg, *rest) = refs
        else:
            (xs, src, dst, za, zd, on, agg, *rest) = refs
            deg = None
        _sc_agg_body(with_deg, xs, src, dst, za, zd, on, agg, deg, *rest)

    return pl.kernel(body, out_type=out_type, mesh=_mesh(),
                     scratch_types=scratch,
                     compiler_params=pltpu.CompilerParams(
                         use_tc_tiling_on_sc=False),
                     name="sc_agg_deg" if with_deg else "sc_agg")


RB = 2000


def _neigh_term(agg_ref, wn_ref):
    r = None
    for cc in range(NC):
        blk = agg_ref[cc]
        for ch in range(NCH):
            q = NCH * cc + ch
            t = jnp.dot(blk[:, ch * DC:(ch + 1) * DC],
                        wn_ref[q * DC:(q + 1) * DC, :],
                        preferred_element_type=jnp.float32)
            r = t if r is None else r + t
    return r


def _tc_self_body(x_ref, ws_ref, b_ref, t_ref):
    t_ref[...] = (jnp.dot(x_ref[...], ws_ref[...],
                          preferred_element_type=jnp.float32)
                  + b_ref[...][None, :])


def _tc_combine1_body(t_ref, agg_ref, deg_ref, wn_ref, h_ref):
    r = _neigh_term(agg_ref, wn_ref)
    deg = jnp.maximum(deg_ref[:, 0:1], 1.0)
    h_ref[...] = jnp.maximum(t_ref[...] + r / deg, 0.0)


def _tc_combine2_body(t_ref, agg_ref, deg_ref, wn_ref, out_ref):
    r = _neigh_term(agg_ref, wn_ref)
    deg = jnp.maximum(deg_ref[:, 0:1], 1.0)
    z = t_ref[...] + r / deg
    m = jnp.max(z, axis=1, keepdims=True)
    zm = z - m
    out_ref[...] = zm - jnp.log(jnp.sum(jnp.exp(zm), axis=1, keepdims=True))


def _tc_self(x, Ws, b, name):
    return pl.pallas_call(
        _tc_self_body,
        grid=(N // RB,),
        in_specs=[
            pl.BlockSpec((RB, D), lambda i: (i, 0)),
            pl.BlockSpec((D, D), lambda i: (0, 0)),
            pl.BlockSpec((D,), lambda i: (0,)),
        ],
        out_specs=pl.BlockSpec((RB, D), lambda i: (i, 0)),
        out_shape=jax.ShapeDtypeStruct((N, D), jnp.float32),
        name=name,
    )(x, Ws, b)


def _tc_combine(body, t, agg4, deg16, Wn, name):
    return pl.pallas_call(
        body,
        grid=(N // RB,),
        in_specs=[
            pl.BlockSpec((RB, D), lambda i: (i, 0)),
            pl.BlockSpec((NC, RB, NCH * DC), lambda i: (0, i, 0)),
            pl.BlockSpec((RB, DEGW), lambda i: (i, 0)),
            pl.BlockSpec((D, D), lambda i: (0, 0)),
        ],
        out_specs=pl.BlockSpec((RB, D), lambda i: (i, 0)),
        out_shape=jax.ShapeDtypeStruct((N, D), jnp.float32),
        name=name,
    )(t, agg4, deg16, Wn)


def kernel(x, W_self1, W_neigh1, b1, W_self2, W_neigh2, b2, edge_index):
    src = edge_index[0].astype(jnp.int32)
    dst = edge_index[1].astype(jnp.int32)
    src4 = ((src * NQ)[None, :]
            + jnp.arange(NQ, dtype=jnp.int32)[:, None]).reshape(NQ * NS, G, B)
    dstr = dst.reshape(NS, G, B)
    xsplit = x.reshape(NQ * N, DC)

    zeros_a = jnp.zeros((ZR, DC), jnp.float32)
    zeros_d = jnp.zeros((RPT, DEGW), jnp.float32)
    ones_d = jnp.ones((B, DEGW), jnp.float32)

    agg1, deg16 = _make_sc_agg(True)(xsplit, src4, dstr, zeros_a, zeros_d,
                                     ones_d)
    t1 = _tc_self(x, W_self1, b1, "tc_self1")
    h = _tc_combine(_tc_combine1_body, t1, agg1, deg16,
                    W_neigh1, "tc_combine1")
    (agg2,) = _make_sc_agg(False)(h.reshape(NQ * N, DC), src4, dstr,
                                  zeros_a, zeros_d, ones_d)
    t2 = _tc_self(h, W_self2, b2, "tc_self2")
    return _tc_combine(_tc_combine2_body, t2, agg2,
                       deg16, W_neigh2, "tc_combine2")


# speedup vs baseline: 8.0618x; 1.0541x over previous
import functools

import jax
import jax.numpy as jnp
from jax import lax
from jax.experimental import pallas as pl
from jax.experimental.pallas import tpu as pltpu
from jax.experimental.pallas import tpu_sc as plsc

N = 10000
E = 160000
D = 256
DC = 64
NCH = 2
NQ = 4
NC = 2
NS = 16
EPT = E // NS
B = 125
G = EPT // B
DEGW = 16
NP = 10000
RPT = NP // NS
ZR = 125


@functools.cache
def _mesh():
    return plsc.VectorSubcoreMesh(core_axis_name="c", subcore_axis_name="s",
                                  num_cores=NC, num_subcores=NS)


NBUF = 5


def _sc_agg_body(with_deg, xs_hbm, src_hbm, dst_hbm, za_hbm, zd_hbm, on_hbm,
                 agg_hbm, deg_hbm, src_v, dst_v, *rest):
    rows = rest[:NBUF]
    ones_v, agg_s, deg_s = rest[NBUF:NBUF + 3]
    sg = rest[NBUF + 3:2 * NBUF + 3]
    ss = rest[2 * NBUF + 3:3 * NBUF + 3]
    sd = rest[3 * NBUF + 3]
    c = lax.axis_index("c")
    s = lax.axis_index("s")

    pltpu.sync_copy(dst_hbm.at[s], dst_v)
    if with_deg:
        @pl.when(c == 0)
        def _():
            pltpu.sync_copy(zd_hbm, deg_s.at[pl.ds(s * RPT, RPT)])
            pltpu.sync_copy(on_hbm, ones_v)

    def gather(g, rows, sem):
        return pltpu.make_async_copy(xs_hbm.at[src_v.at[g]], rows, sem)

    def scatter(g, rows, sem):
        return pltpu.make_async_copy(rows, agg_s.at[dst_v.at[g]], sem)

    for ch in range(NCH):
        q = NCH * c + ch
        pltpu.sync_copy(src_hbm.at[q * NS + s], src_v)

        deg_pass = with_deg and ch == 0

        def deg_add(g):
            return pltpu.make_async_copy(ones_v, deg_s.at[dst_v.at[g]], sd)

        for k in range(NBUF):
            gather(k, rows[k], sg[k]).start()

        if ch > 0:
            pltpu.sync_copy(agg_s.at[pl.ds(s * RPT, RPT)],
                            agg_hbm.at[c, pl.ds(s * RPT, RPT),
                                       pl.ds((ch - 1) * DC, DC)])
        for j in range(RPT // ZR):
            pltpu.sync_copy(za_hbm, agg_s.at[pl.ds(s * RPT + j * ZR, ZR)])
        plsc.subcore_barrier()

        def stage(i, _, deg_pass=deg_pass):
            base = NBUF * i
            for k in range(NBUF):
                g = base + k
                gather(g, rows[k], sg[k]).wait()
                scatter(g, rows[k], ss[k]).start(add=True)
            if deg_pass:
                @pl.when(c == 0)
                def _():
                    @pl.when(i > 0)
                    def _():
                        for k in range(NBUF):
                            deg_add(base + k).wait()
                    for k in range(NBUF):
                        deg_add(base + k).start(add=True)

            @pl.when(i + 1 < G // NBUF)
            def _():
                for k in range(NBUF):
                    scatter(base + k, rows[k], ss[k]).wait()
                    gather(base + NBUF + k, rows[k], sg[k]).start()
            return _

        lax.fori_loop(0, G // NBUF, stage, None)
        for k in range(NBUF):
            scatter(G - NBUF + k, rows[k], ss[k]).wait()
        if deg_pass:
            @pl.when(c == 0)
            def _():
                for k in range(NBUF):
                    deg_add(0).wait()
        plsc.subcore_barrier()

    pltpu.sync_copy(agg_s.at[pl.ds(s * RPT, RPT)],
                    agg_hbm.at[c, pl.ds(s * RPT, RPT),
                               pl.ds((NCH - 1) * DC, DC)])
    if with_deg:
        @pl.when(jnp.logical_and(c == 0, s == 0))
        def _():
            pltpu.sync_copy(deg_s.at[pl.ds(0, N)], deg_hbm)


@functools.cache
def _make_sc_agg(with_deg):
    out_type = [jax.ShapeDtypeStruct((NC, NP, NCH * DC), jnp.float32)]
    if with_deg:
        out_type.append(jax.ShapeDtypeStruct((N, DEGW), jnp.float32))
    scratch = (
        [pltpu.VMEM((G, B), jnp.int32)] * 2
        + [pltpu.VMEM((B, DC), jnp.float32)] * NBUF
        + [pltpu.VMEM((B, DEGW), jnp.float32)]
        + [pltpu.VMEM_SHARED((NP, DC), jnp.float32)]
        + [pltpu.VMEM_SHARED((NP, DEGW), jnp.float32)]
        + [pltpu.SemaphoreType.DMA] * (2 * NBUF + 1)
    )

    def body(*refs):
        if with_deg:
            (xs, src, dst, za, zd, on, agg, deg, *rest) = refs
        else:
            (xs, src, dst, za, zd, on, agg, *rest) = refs
            deg = None
        _sc_agg_body(with_deg, xs, src, dst, za, zd, on, agg, deg, *rest)

    return pl.kernel(body, out_type=out_type, mesh=_mesh(),
                     scratch_types=scratch,
                     compiler_params=pltpu.CompilerParams(
                         use_tc_tiling_on_sc=False),
                     name="sc_agg_deg" if with_deg else "sc_agg")


RB = 2000


def _neigh_term(agg_ref, wn_ref):
    r = None
    for cc in range(NC):
        blk = agg_ref[cc]
        for ch in range(NCH):
            q = NCH * cc + ch
            t = jnp.dot(blk[:, ch * DC:(ch + 1) * DC],
                        wn_ref[q * DC:(q + 1) * DC, :],
                        preferred_element_type=jnp.float32)
            r = t if r is None else r + t
    return r


def _tc_self_body(x_ref, ws_ref, b_ref, t_ref):
    t_ref[...] = (jnp.dot(x_ref[...], ws_ref[...],
                          preferred_element_type=jnp.float32)
                  + b_ref[...][None, :])


def _tc_combine1_body(t_ref, agg_ref, deg_ref, wn_ref, h_ref):
    r = _neigh_term(agg_ref, wn_ref)
    deg = jnp.maximum(deg_ref[:, 0:1], 1.0)
    h_ref[...] = jnp.maximum(t_ref[...] + r / deg, 0.0)


def _tc_combine2_body(t_ref, agg_ref, deg_ref, wn_ref, out_ref):
    r = _neigh_term(agg_ref, wn_ref)
    deg = jnp.maximum(deg_ref[:, 0:1], 1.0)
    z = t_ref[...] + r / deg
    m = jnp.max(z, axis=1, keepdims=True)
    zm = z - m
    out_ref[...] = zm - jnp.log(jnp.sum(jnp.exp(zm), axis=1, keepdims=True))


def _tc_self(x, Ws, b, name):
    return pl.pallas_call(
        _tc_self_body,
        grid=(N // RB,),
        in_specs=[
            pl.BlockSpec((RB, D), lambda i: (i, 0)),
            pl.BlockSpec((D, D), lambda i: (0, 0)),
            pl.BlockSpec((D,), lambda i: (0,)),
        ],
        out_specs=pl.BlockSpec((RB, D), lambda i: (i, 0)),
        out_shape=jax.ShapeDtypeStruct((N, D), jnp.float32),
        name=name,
    )(x, Ws, b)


def _tc_combine(body, t, agg4, deg16, Wn, name):
    return pl.pallas_call(
        body,
        grid=(N // RB,),
        in_specs=[
            pl.BlockSpec((RB, D), lambda i: (i, 0)),
            pl.BlockSpec((NC, RB, NCH * DC), lambda i: (0, i, 0)),
            pl.BlockSpec((RB, DEGW), lambda i: (i, 0)),
            pl.BlockSpec((D, D), lambda i: (0, 0)),
        ],
        out_specs=pl.BlockSpec((RB, D), lambda i: (i, 0)),
        out_shape=jax.ShapeDtypeStruct((N, D), jnp.float32),
        name=name,
    )(t, agg4, deg16, Wn)


def kernel(x, W_self1, W_neigh1, b1, W_self2, W_neigh2, b2, edge_index):
    src = edge_index[0].astype(jnp.int32)
    dst = edge_index[1].astype(jnp.int32)
    srcr = src.reshape(NS, G, B)
    src4 = jnp.concatenate([srcr * NQ + q for q in range(NQ)], axis=0)
    dstr = dst.reshape(NS, G, B)
    xsplit = x.reshape(NQ * N, DC)

    zeros_a = jnp.zeros((ZR, DC), jnp.float32)
    zeros_d = jnp.zeros((RPT, DEGW), jnp.float32)
    ones_d = jnp.ones((B, DEGW), jnp.float32)

    agg1, deg16 = _make_sc_agg(True)(xsplit, src4, dstr, zeros_a, zeros_d,
                                     ones_d)
    t1 = _tc_self(x, W_self1, b1, "tc_self1")
    h = _tc_combine(_tc_combine1_body, t1, agg1, deg16,
                    W_neigh1, "tc_combine1")
    (agg2,) = _make_sc_agg(False)(h.reshape(NQ * N, DC), src4, dstr,
                                  zeros_a, zeros_d, ones_d)
    t2 = _tc_self(h, W_self2, b2, "tc_self2")
    return _tc_combine(_tc_combine2_body, t2, agg2,
                       deg16, W_neigh2, "tc_combine2")
